# Initial kernel scaffold; baseline (speedup 1.0000x reference)
import jax
import jax.numpy as jnp
from jax.experimental import pallas as pl


def kernel(x, edge_index, Lat, Lon, W1, b1, W2, b2, Lw, Lb, Lw1, Lb1):
    raise NotImplementedError("write your pallas kernel here")


# speedup vs baseline: 3.4704x; 3.4704x over previous
import functools
import math

import jax
import jax.numpy as jnp
from jax.experimental import pallas as pl
from jax.experimental.pallas import tpu as pltpu

C = 69
H = 121
Wd = 240
NG = H * Wd
NM = 40962
N = NG + NM
HID = 256
OUT_CH = 69
FACTOR = 4
D1 = 80
BN = 512

_SQRT2 = math.sqrt(2.0)


def _gelu(x):
    return x * 0.5 * (1.0 + jax.lax.erf(x / _SQRT2))


def _combine_dec_body(lw_ref, lw1_ref, lb_ref, lb1_ref, lc_ref, lbc_ref):
    lc_ref[...] = jnp.dot(lw_ref[...], lw1_ref[...],
                          preferred_element_type=jnp.float32)
    lbc_ref[...] = jnp.dot(lb_ref[...], lw1_ref[...],
                           preferred_element_type=jnp.float32) + lb1_ref[...]


def _scale_grid_body(xrow_ref, deg_ref, xs_ref, dis_ref):
    dis = jax.lax.rsqrt(deg_ref[...])
    dis_ref[...] = dis
    xs_ref[...] = xrow_ref[...] * dis


def _mesh_assemble_body(g_ref, w4_ref, deg_ref, xs_ref, dis_ref):
    g = g_ref[...]
    w = w4_ref[...]
    mesh = (g[:, 0 * D1:1 * D1] * w[:, 0:1] + g[:, 1 * D1:2 * D1] * w[:, 1:2]
            + g[:, 2 * D1:3 * D1] * w[:, 2:3] + g[:, 3 * D1:4 * D1] * w[:, 3:4])
    dis = jax.lax.rsqrt(deg_ref[...])
    dis_ref[...] = dis
    xs_ref[...] = mesh * dis


def _gcn1_body(a1_ref, xs_ref, dis_ref, w1_ref, b1_ref, gs_ref):
    out1 = dis_ref[...] * (a1_ref[...] + xs_ref[...])
    h = jnp.dot(out1, w1_ref[...], preferred_element_type=jnp.float32) + b1_ref[...]
    gs_ref[...] = _gelu(h) * dis_ref[...]


def _dec_body(a2_ref, gsm_ref, dism_ref, w2_ref, b2_ref, lc_ref, lbc_ref, o_ref):
    out2 = dism_ref[...] * (a2_ref[...] + gsm_ref[...])
    h2 = jnp.dot(out2, w2_ref[...], preferred_element_type=jnp.float32) + b2_ref[...]
    o = jnp.dot(h2, lc_ref[...], preferred_element_type=jnp.float32) + lbc_ref[...]
    o_ref[...] = _gelu(o)


def _row_blocks(n_rows, width):
    grid = (pl.cdiv(n_rows, BN),)
    blk = pl.BlockSpec((BN, width), lambda i: (i, 0))
    return grid, blk


def _full(shape):
    return pl.BlockSpec(shape, lambda i: (0,) * len(shape))


def _tap_indices(Lat, Lon):
    i = Lat - 1
    j = Lon - 1
    fy = (i.astype(jnp.float32) + 0.5) / FACTOR - 0.5
    fx = (j.astype(jnp.float32) + 0.5) / FACTOR - 0.5
    y0 = jnp.floor(fy).astype(jnp.int32)
    x0 = jnp.floor(fx).astype(jnp.int32)
    ty = fy - y0.astype(jnp.float32)
    tx = fx - x0.astype(jnp.float32)
    y0c = jnp.clip(y0, 0, H - 2)
    y1c = jnp.clip(y0 + 1, 0, H - 2)
    x0c = jnp.clip(x0, 0, Wd - 1)
    x1c = jnp.clip(x0 + 1, 0, Wd - 1)
    pole = i == (H - 1) * FACTOR
    pole_idx = (H - 1) * Wd
    idx = jnp.stack([
        jnp.where(pole, pole_idx, y0c * Wd + x0c),
        jnp.where(pole, pole_idx, y0c * Wd + x1c),
        jnp.where(pole, pole_idx, y1c * Wd + x0c),
        jnp.where(pole, pole_idx, y1c * Wd + x1c),
    ])
    one = jnp.ones_like(ty)
    zero = jnp.zeros_like(ty)
    w4 = jnp.stack([
        jnp.where(pole, one, (1 - ty) * (1 - tx)),
        jnp.where(pole, zero, (1 - ty) * tx),
        jnp.where(pole, zero, ty * (1 - tx)),
        jnp.where(pole, zero, ty * tx),
    ], axis=1)
    return idx, w4


def kernel(x, edge_index, Lat, Lon, W1, b1, W2, b2, Lw, Lb, Lw1, Lb1):
    src = edge_index[0]
    dst = edge_index[1]

    xrow = jnp.pad(x[0].reshape(C, NG).T, ((0, 0), (0, D1 - C)))
    W1p = jnp.pad(W1, ((0, D1 - C), (0, 0)))
    b1r = b1.reshape(1, HID)
    b2r = b2.reshape(1, HID)
    lbr = Lb.reshape(1, HID)
    lb1r = Lb1.reshape(1, OUT_CH)
    idx4, w4 = _tap_indices(Lat, Lon)

    deg = jnp.zeros((N,), jnp.float32).at[dst].add(1.0) + 1.0
    deg_g = deg[:NG].reshape(NG, 1)
    deg_m = deg[NG:].reshape(NM, 1)
    gtap = jnp.take(xrow, idx4.reshape(-1), axis=0).reshape(4, NM, D1)
    gtap = jnp.concatenate([gtap[0], gtap[1], gtap[2], gtap[3]], axis=1)

    lc, lbc = pl.pallas_call(
        _combine_dec_body,
        out_shape=(jax.ShapeDtypeStruct((HID, OUT_CH), jnp.float32),
                   jax.ShapeDtypeStruct((1, OUT_CH), jnp.float32)),
    )(Lw, Lw1, lbr, lb1r)

    grid_g, blk_g = _row_blocks(NG, D1)
    xs_g, dis_g = pl.pallas_call(
        _scale_grid_body,
        grid=grid_g,
        in_specs=[blk_g, pl.BlockSpec((BN, 1), lambda i: (i, 0))],
        out_specs=(blk_g, pl.BlockSpec((BN, 1), lambda i: (i, 0))),
        out_shape=(jax.ShapeDtypeStruct((NG, D1), jnp.float32),
                   jax.ShapeDtypeStruct((NG, 1), jnp.float32)),
    )(xrow, deg_g)

    grid_m, blk_m = _row_blocks(NM, D1)
    xs_m, dis_m = pl.pallas_call(
        _mesh_assemble_body,
        grid=grid_m,
        in_specs=[pl.BlockSpec((BN, 4 * D1), lambda i: (i, 0)),
                  pl.BlockSpec((BN, 4), lambda i: (i, 0)),
                  pl.BlockSpec((BN, 1), lambda i: (i, 0))],
        out_specs=(blk_m, pl.BlockSpec((BN, 1), lambda i: (i, 0))),
        out_shape=(jax.ShapeDtypeStruct((NM, D1), jnp.float32),
                   jax.ShapeDtypeStruct((NM, 1), jnp.float32)),
    )(gtap, w4, deg_m)

    xs = jnp.concatenate([xs_g, xs_m], axis=0)
    dis = jnp.concatenate([dis_g, dis_m], axis=0)

    a1 = jnp.zeros((N, D1), jnp.float32).at[dst].add(jnp.take(xs, src, axis=0))

    grid_n, blk_n = _row_blocks(N, D1)
    gs = pl.pallas_call(
        _gcn1_body,
        grid=grid_n,
        in_specs=[blk_n, blk_n, pl.BlockSpec((BN, 1), lambda i: (i, 0)),
                  _full((D1, HID)), _full((1, HID))],
        out_specs=pl.BlockSpec((BN, HID), lambda i: (i, 0)),
        out_shape=jax.ShapeDtypeStruct((N, HID), jnp.float32),
    )(a1, xs, dis, W1p, b1r)

    a2 = jnp.zeros((N, HID), jnp.float32).at[dst].add(jnp.take(gs, src, axis=0))
    a2m = a2[NG:]
    gsm = gs[NG:]

    grid_d, blk_d = _row_blocks(NM, HID)
    o = pl.pallas_call(
        _dec_body,
        grid=grid_d,
        in_specs=[blk_d, blk_d, pl.BlockSpec((BN, 1), lambda i: (i, 0)),
                  _full((HID, HID)), _full((1, HID)),
                  _full((HID, OUT_CH)), _full((1, OUT_CH))],
        out_specs=pl.BlockSpec((BN, OUT_CH), lambda i: (i, 0)),
        out_shape=jax.ShapeDtypeStruct((NM, OUT_CH), jnp.float32),
    )(a2m, gsm, dis_m, W2, b2r, lc, lbc)

    return o.T[None]
